# baseline (speedup 1.0000x reference)
import functools

import jax
import jax.numpy as jnp
from jax import lax
from jax.experimental import pallas as pl
from jax.experimental.pallas import tpu as pltpu
from jax.experimental.pallas import tpu_sc as plsc

_NC = 2
_NS = 16
_L = 16
_NW = _NC * _NS
_CHUNK = 128


@functools.lru_cache(maxsize=None)
def _build(B, D):
    assert B % (_NW * _CHUNK) == 0 and D % _L == 0
    bpw = B // _NW
    nch = bpw // _CHUNK
    nblk = bpw // _L

    mesh = plsc.VectorSubcoreMesh(core_axis_name="c", subcore_axis_name="s")

    @functools.partial(
        pl.kernel,
        mesh=mesh,
        compiler_params=pltpu.CompilerParams(
            needs_layout_passes=False, use_tc_tiling_on_sc=False),
        out_type=jax.ShapeDtypeStruct((B,), jnp.float32),
        scratch_types=[
            pltpu.VMEM((bpw,), jnp.int32),
            pltpu.VMEM((D * bpw,), jnp.float32),
            pltpu.VMEM(((D + 1) * _L,), jnp.float32),
            pltpu.VMEM((bpw,), jnp.float32),
            pltpu.SemaphoreType.DMA,
        ],
    )
    def gather_dot(idx_hbm, aux_hbm, tableT_hbm, out_hbm,
                   idx_v, cols_v, aux_v, res_v, sem):
        wid = lax.axis_index("s") * _NC + lax.axis_index("c")
        base = wid * bpw
        pltpu.sync_copy(idx_hbm.at[pl.ds(base, bpw)], idx_v)
        pltpu.sync_copy(aux_hbm, aux_v)

        prev = None
        for d in range(D):
            cur = [
                pltpu.async_copy(
                    tableT_hbm.at[d].at[idx_v.at[pl.ds(j * _CHUNK, _CHUNK)]],
                    cols_v.at[pl.ds(d * bpw + j * _CHUNK, _CHUNK)], sem)
                for j in range(nch)
            ]
            if prev is not None:
                for h in prev:
                    h.wait()
            prev = cur
        for h in prev:
            h.wait()

        def block(v, carry):
            o = v * _L
            acc = aux_v[pl.ds(D * _L, _L)]
            for d in range(D):
                acc = acc + cols_v[pl.ds(d * bpw + o, _L)] * \
                    aux_v[pl.ds(d * _L, _L)]
            res_v[pl.ds(o, _L)] = acc
            return carry

        lax.fori_loop(0, nblk, block, 0)
        pltpu.sync_copy(res_v, out_hbm.at[pl.ds(base, bpw)])

    return gather_dot


def kernel(x_movie, x_user, movie_table, fc_w, fc_b):
    B = x_movie.shape[0]
    D = movie_table.shape[1]
    wc = fc_w[0, :D] + fc_w[0, D:]
    aux = jnp.broadcast_to(
        jnp.concatenate([wc, fc_b])[:, None], (D + 1, _L)
    ).astype(jnp.float32).reshape(-1)
    idx = x_movie.astype(jnp.int32)
    out = _build(B, D)(idx, aux, movie_table.T)
    return out.reshape(B, 1)


# speedup vs baseline: 19.6588x; 19.6588x over previous
import functools

import jax
import jax.numpy as jnp
from jax import lax
from jax.experimental import pallas as pl
from jax.experimental.pallas import tpu as pltpu
from jax.experimental.pallas import tpu_sc as plsc

_NC = 2
_NS = 16
_L = 16
_NW = _NC * _NS
_TILE = 128


@functools.lru_cache(maxsize=None)
def _build(B, D):
    assert B % (_NW * _L) == 0 and D % _L == 0
    bpw = B // _NW
    ngr = bpw // _L

    mesh = plsc.VectorSubcoreMesh(core_axis_name="c", subcore_axis_name="s")

    @functools.partial(
        pl.kernel,
        mesh=mesh,
        compiler_params=pltpu.CompilerParams(
            needs_layout_passes=False, use_tc_tiling_on_sc=True),
        out_type=jax.ShapeDtypeStruct((B,), jnp.float32),
        scratch_types=[
            pltpu.VMEM((bpw,), jnp.int32),
            pltpu.VMEM((_L, D, _TILE), jnp.float32),
            pltpu.VMEM(((D + 1) * _L,), jnp.float32),
            pltpu.VMEM((bpw,), jnp.float32),
            pltpu.SemaphoreType.DMA,
        ],
    )
    def gather_dot(idx_hbm, aux_hbm, tableT_hbm, out_hbm,
                   idx_v, stage_v, aux_v, res_v, sem):
        wid = lax.axis_index("s") * _NC + lax.axis_index("c")
        base = wid * bpw
        pltpu.sync_copy(idx_hbm.at[pl.ds(base, bpw)], idx_v)
        pltpu.sync_copy(aux_hbm, aux_v)

        lanes16 = lax.iota(jnp.int32, _L)

        def group(g, carry):
            o = g * _L
            vec = idx_v[pl.ds(o, _L)]
            vblk = lax.shift_right_logical(vec, 7)
            copies = [
                pltpu.async_copy(
                    tableT_hbm.at[pl.ds(0, D), pl.ds(vblk[k] * _TILE, _TILE)],
                    stage_v.at[k], sem)
                for k in range(_L)
            ]
            for h in copies:
                h.wait()
            lane = lax.bitwise_and(vec, jnp.int32(127))
            acc = aux_v[pl.ds(D * _L, _L)]
            for d in range(D):
                col = plsc.load_gather(
                    stage_v, [lanes16, jnp.full((_L,), d, jnp.int32), lane])
                acc = acc + col * aux_v[pl.ds(d * _L, _L)]
            res_v[pl.ds(o, _L)] = acc
            return carry

        lax.fori_loop(0, ngr, group, 0)
        pltpu.sync_copy(res_v, out_hbm.at[pl.ds(base, bpw)])

    return gather_dot


def kernel(x_movie, x_user, movie_table, fc_w, fc_b):
    B = x_movie.shape[0]
    D = movie_table.shape[1]
    wc = fc_w[0, :D] + fc_w[0, D:]
    aux = jnp.broadcast_to(
        jnp.concatenate([wc, fc_b])[:, None], (D + 1, _L)
    ).astype(jnp.float32).reshape(-1)
    idx = x_movie.astype(jnp.int32)
    out = _build(B, D)(idx, aux, movie_table.T)
    return out.reshape(B, 1)


# speedup vs baseline: 29.0836x; 1.4794x over previous
import functools

import jax
import jax.numpy as jnp
from jax import lax
from jax.experimental import pallas as pl
from jax.experimental.pallas import tpu as pltpu
from jax.experimental.pallas import tpu_sc as plsc

_NC = 2
_NS = 16
_L = 16
_NW = _NC * _NS
_CHUNK = 128
_CW = 16384


def _dense_scores(tableT, wcb):
    D, V = tableT.shape

    def body(t_ref, w_ref, s_ref):
        x = t_ref[...]
        w = w_ref[:, 0:1]
        s_ref[...] = jnp.sum(x * w, axis=0)

    return pl.pallas_call(
        body,
        grid=(pl.cdiv(V, _CW),),
        in_specs=[
            pl.BlockSpec((D, _CW), lambda i: (0, i)),
            pl.BlockSpec((D, 128), lambda i: (0, 0)),
        ],
        out_specs=pl.BlockSpec((_CW,), lambda i: (i,)),
        out_shape=jax.ShapeDtypeStruct((V,), jnp.float32),
    )(tableT, wcb)


@functools.lru_cache(maxsize=None)
def _build_pick(B, V):
    assert B % (_NW * _CHUNK) == 0
    bpw = B // _NW
    nch = bpw // _CHUNK

    mesh = plsc.VectorSubcoreMesh(core_axis_name="c", subcore_axis_name="s")

    @functools.partial(
        pl.kernel,
        mesh=mesh,
        compiler_params=pltpu.CompilerParams(
            needs_layout_passes=False, use_tc_tiling_on_sc=False),
        out_type=jax.ShapeDtypeStruct((B,), jnp.float32),
        scratch_types=[
            pltpu.VMEM((nch, _CHUNK), jnp.int32),
            pltpu.VMEM((bpw,), jnp.float32),
            pltpu.VMEM((_L,), jnp.float32),
            pltpu.SemaphoreType.DMA,
        ],
    )
    def pick(idx_hbm, bias_hbm, s_hbm, out_hbm, idx_v, val_v, bias_v, sem):
        wid = lax.axis_index("s") * _NC + lax.axis_index("c")
        base = wid * bpw
        pltpu.sync_copy(idx_hbm.at[wid], idx_v)
        pltpu.sync_copy(bias_hbm, bias_v)
        copies = [
            pltpu.async_copy(
                s_hbm.at[idx_v.at[j]],
                val_v.at[pl.ds(j * _CHUNK, _CHUNK)], sem)
            for j in range(nch)
        ]
        for h in copies:
            h.wait()
        bias = bias_v[...]
        for j in range(bpw // _L):
            o = j * _L
            val_v[pl.ds(o, _L)] = val_v[pl.ds(o, _L)] + bias
        pltpu.sync_copy(val_v, out_hbm.at[pl.ds(base, bpw)])

    return pick


def kernel(x_movie, x_user, movie_table, fc_w, fc_b):
    B = x_movie.shape[0]
    V, D = movie_table.shape
    wc = fc_w[0, :D] + fc_w[0, D:]
    wcb = jnp.broadcast_to(wc[:, None], (D, 128))
    bias = jnp.broadcast_to(fc_b, (_L,)).astype(jnp.float32)
    idx = x_movie.astype(jnp.int32).reshape(_NW, B // (_NW * _CHUNK), _CHUNK)
    s = _dense_scores(movie_table.T, wcb)
    out = _build_pick(B, V)(idx, bias, s)
    return out.reshape(B, 1)


# speedup vs baseline: 36.1630x; 1.2434x over previous
import functools

import jax
import jax.numpy as jnp
from jax import lax
from jax.experimental import pallas as pl
from jax.experimental.pallas import tpu as pltpu
from jax.experimental.pallas import tpu_sc as plsc

_NC = 2
_NS = 16
_L = 16
_NW = _NC * _NS
_CHUNK = 128
_CW = 32768


def _dense_scores(tableT, wcb):
    D, V = tableT.shape

    def body(t_ref, w_ref, s_ref):
        x = t_ref[...]
        w = w_ref[:, 0:1]
        s_ref[...] = jnp.sum(x * w, axis=0)

    return pl.pallas_call(
        body,
        grid=(pl.cdiv(V, _CW),),
        compiler_params=pltpu.CompilerParams(
            dimension_semantics=("arbitrary",)),
        in_specs=[
            pl.BlockSpec((D, _CW), lambda i: (0, i)),
            pl.BlockSpec((D, 128), lambda i: (0, 0)),
        ],
        out_specs=pl.BlockSpec((_CW,), lambda i: (i,)),
        out_shape=jax.ShapeDtypeStruct((V,), jnp.float32),
    )(tableT, wcb)


@functools.lru_cache(maxsize=None)
def _build_pick(B, V):
    assert B % (_NW * _CHUNK) == 0
    bpw = B // _NW
    nch = bpw // _CHUNK

    mesh = plsc.VectorSubcoreMesh(core_axis_name="c", subcore_axis_name="s")

    @functools.partial(
        pl.kernel,
        mesh=mesh,
        compiler_params=pltpu.CompilerParams(
            needs_layout_passes=False, use_tc_tiling_on_sc=False),
        out_type=jax.ShapeDtypeStruct((B,), jnp.float32),
        scratch_types=[
            pltpu.VMEM((nch, _CHUNK), jnp.int32),
            pltpu.VMEM((bpw,), jnp.float32),
            pltpu.VMEM((_L,), jnp.float32),
            pltpu.SemaphoreType.DMA,
        ],
    )
    def pick(idx_hbm, bias_hbm, s_hbm, out_hbm, idx_v, val_v, bias_v, sem):
        wid = lax.axis_index("s") * _NC + lax.axis_index("c")
        base = wid * bpw
        pltpu.sync_copy(idx_hbm.at[wid], idx_v)
        pltpu.sync_copy(bias_hbm, bias_v)
        copies = [
            pltpu.async_copy(
                s_hbm.at[idx_v.at[j]],
                val_v.at[pl.ds(j * _CHUNK, _CHUNK)], sem)
            for j in range(nch)
        ]
        for h in copies:
            h.wait()
        bias = bias_v[...]
        for j in range(bpw // _L):
            o = j * _L
            val_v[pl.ds(o, _L)] = val_v[pl.ds(o, _L)] + bias
        pltpu.sync_copy(val_v, out_hbm.at[pl.ds(base, bpw)])

    return pick


def kernel(x_movie, x_user, movie_table, fc_w, fc_b):
    B = x_movie.shape[0]
    V, D = movie_table.shape
    wc = fc_w[0, :D] + fc_w[0, D:]
    wcb = jnp.broadcast_to(wc[:, None], (D, 128))
    bias = jnp.broadcast_to(fc_b, (_L,)).astype(jnp.float32)
    idx = x_movie.astype(jnp.int32).reshape(_NW, B // (_NW * _CHUNK), _CHUNK)
    s = _dense_scores(movie_table.T, wcb)
    out = _build_pick(B, V)(idx, bias, s)
    return out.reshape(B, 1)


# speedup vs baseline: 39.8505x; 1.1020x over previous
import functools

import jax
import jax.numpy as jnp
from jax import lax
from jax.experimental import pallas as pl
from jax.experimental.pallas import tpu as pltpu
from jax.experimental.pallas import tpu_sc as plsc

_NC = 2
_NS = 16
_L = 16
_NW = _NC * _NS
_CHUNK = 128
_CW = 65536


def _dense_scores(tableT, wcb):
    D, V = tableT.shape

    def body(t_ref, w_ref, s_ref):
        x = t_ref[...]
        w = w_ref[:, 0:1]
        s_ref[...] = jnp.sum(x * w, axis=0)

    return pl.pallas_call(
        body,
        grid=(pl.cdiv(V, _CW),),
        compiler_params=pltpu.CompilerParams(
            dimension_semantics=("arbitrary",)),
        in_specs=[
            pl.BlockSpec((D, _CW), lambda i: (0, i)),
            pl.BlockSpec((D, 128), lambda i: (0, 0)),
        ],
        out_specs=pl.BlockSpec((_CW,), lambda i: (i,)),
        out_shape=jax.ShapeDtypeStruct((V,), jnp.float32),
    )(tableT, wcb)


@functools.lru_cache(maxsize=None)
def _build_pick(B, V):
    assert B % (_NW * _CHUNK) == 0
    bpw = B // _NW
    nch = bpw // _CHUNK

    mesh = plsc.VectorSubcoreMesh(core_axis_name="c", subcore_axis_name="s")

    @functools.partial(
        pl.kernel,
        mesh=mesh,
        compiler_params=pltpu.CompilerParams(
            needs_layout_passes=False, use_tc_tiling_on_sc=False),
        out_type=jax.ShapeDtypeStruct((B,), jnp.float32),
        scratch_types=[
            pltpu.VMEM((nch, _CHUNK), jnp.int32),
            pltpu.VMEM((bpw,), jnp.float32),
            pltpu.VMEM((_L,), jnp.float32),
            pltpu.SemaphoreType.DMA,
        ],
    )
    def pick(idx_hbm, bias_hbm, s_hbm, out_hbm, idx_v, val_v, bias_v, sem):
        wid = lax.axis_index("s") * _NC + lax.axis_index("c")
        base = wid * bpw
        pltpu.sync_copy(idx_hbm.at[wid], idx_v)
        pltpu.sync_copy(bias_hbm, bias_v)
        copies = [
            pltpu.async_copy(
                s_hbm.at[idx_v.at[j]],
                val_v.at[pl.ds(j * _CHUNK, _CHUNK)], sem)
            for j in range(nch)
        ]
        for h in copies:
            h.wait()
        bias = bias_v[...]
        for j in range(bpw // _L):
            o = j * _L
            val_v[pl.ds(o, _L)] = val_v[pl.ds(o, _L)] + bias
        pltpu.sync_copy(val_v, out_hbm.at[pl.ds(base, bpw)])

    return pick


def kernel(x_movie, x_user, movie_table, fc_w, fc_b):
    B = x_movie.shape[0]
    V, D = movie_table.shape
    wc = fc_w[0, :D] + fc_w[0, D:]
    wcb = jnp.broadcast_to(wc[:, None], (D, 128))
    bias = jnp.broadcast_to(fc_b, (_L,)).astype(jnp.float32)
    idx = x_movie.astype(jnp.int32).reshape(_NW, B // (_NW * _CHUNK), _CHUNK)
    s = _dense_scores(movie_table.T, wcb)
    out = _build_pick(B, V)(idx, bias, s)
    return out.reshape(B, 1)
